# Initial kernel scaffold; baseline (speedup 1.0000x reference)
import jax
import jax.numpy as jnp
from jax.experimental import pallas as pl


def kernel(feat, edge_index, W_self, b_self, W_neigh, b_neigh):
    raise NotImplementedError("write your pallas kernel here")


# speedup vs baseline: 1.8558x; 1.8558x over previous
import jax
import jax.numpy as jnp
from jax import lax
from jax.experimental import pallas as pl
from jax.experimental.pallas import tpu as pltpu
from jax.experimental.pallas import tpu_sc as plsc

N = 10000
NH = 3336
NPASS = 3
E = 160000
D = 256
DH = D // 2
NTILES = 16
EPT = E // NTILES
CHUNK = 128
NCHUNK = 80
EPT_PAD = NCHUNK * CHUNK
NBUF = 4
TRASH = 64
AROWS = NH + TRASH
ROWS_PT = 208
ROWS_TAIL = AROWS - NTILES * ROWS_PT
WB_TAIL = NH - NTILES * ROWS_PT
DEGW = 128
LPC = CHUNK // 16

DROWS = N + TRASH
DROWS_PT = 624
DROWS_TAIL = DROWS - NTILES * DROWS_PT
DWB_TAIL = N - NTILES * DROWS_PT


def _sc_agg_body(feat_lo, feat_hi, src_h, dst_h, zf,
                 out_lo, out_hi,
                 src_v, dst_v, dstq_v, rows_v, acc_s,
                 sem0, sem1, sem2, sem3):
  cid = lax.axis_index("c")
  sid = lax.axis_index("s")
  sems = [sem0, sem1, sem2, sem3]

  pltpu.sync_copy(src_h.at[sid], src_v)
  pltpu.sync_copy(dst_h.at[sid], dst_v)

  r0 = pl.multiple_of(sid * ROWS_PT, 8)
  t0 = NTILES * ROWS_PT
  last = sid == NTILES - 1
  trash_vec = NH + sid * 4 + (lax.iota(jnp.int32, 16) % 4)

  for q in range(NPASS):
    lo = q * NH

    def remap(i, carry):
      for k in range(LPC):
        d = dst_v[i, pl.ds(k * 16, 16)]
        inr = (d >= lo) & (d < lo + NH)
        dstq_v[i, pl.ds(k * 16, 16)] = jnp.where(inr, d - lo, trash_vec)
      return carry

    lax.fori_loop(0, NCHUNK, remap, 0)

    pltpu.sync_copy(zf, acc_s.at[pl.ds(r0, ROWS_PT)])

    @pl.when(last)
    def _():
      pltpu.sync_copy(zf.at[pl.ds(0, ROWS_TAIL)],
                      acc_s.at[pl.ds(t0, ROWS_TAIL)])

    plsc.subcore_barrier()

    def start_gather(j, b):
      @pl.when(cid == 0)
      def _():
        pltpu.async_copy(feat_lo.at[src_v.at[j]], rows_v.at[b], sems[b])

      @pl.when(cid != 0)
      def _():
        pltpu.async_copy(feat_hi.at[src_v.at[j]], rows_v.at[b], sems[b])

    def wait_gather(j, b):
      @pl.when(cid == 0)
      def _():
        pltpu.make_async_copy(feat_lo.at[src_v.at[j]], rows_v.at[b],
                              sems[b]).wait()

      @pl.when(cid != 0)
      def _():
        pltpu.make_async_copy(feat_hi.at[src_v.at[j]], rows_v.at[b],
                              sems[b]).wait()

    for b in range(NBUF):
      start_gather(b, b)

    def outer(g, carry):
      for b in range(NBUF):
        j = g * NBUF + b
        wait_gather(j, b)
        pltpu.sync_copy(rows_v.at[b], acc_s.at[dstq_v.at[j]], add=True)
        jn = j + NBUF

        @pl.when(jn < NCHUNK)
        def _():
          start_gather(jn, b)
      return carry

    lax.fori_loop(0, NCHUNK // NBUF, outer, 0)

    plsc.subcore_barrier()

    o0 = pl.multiple_of(lo + sid * ROWS_PT, 8)
    ot = pl.multiple_of(lo + t0, 8)

    has_tail = lo + NTILES * ROWS_PT + WB_TAIL <= N

    @pl.when(cid == 0)
    def _():
      pltpu.sync_copy(acc_s.at[pl.ds(r0, ROWS_PT)],
                      out_lo.at[pl.ds(o0, ROWS_PT)])

      if has_tail:
        @pl.when(last)
        def _():
          pltpu.sync_copy(acc_s.at[pl.ds(t0, WB_TAIL)],
                          out_lo.at[pl.ds(ot, WB_TAIL)])

    @pl.when(cid != 0)
    def _():
      pltpu.sync_copy(acc_s.at[pl.ds(r0, ROWS_PT)],
                      out_hi.at[pl.ds(o0, ROWS_PT)])

      if has_tail:
        @pl.when(last)
        def _():
          pltpu.sync_copy(acc_s.at[pl.ds(t0, WB_TAIL)],
                          out_hi.at[pl.ds(ot, WB_TAIL)])


_sc_agg = pl.kernel(
    _sc_agg_body,
    out_type=(
        jax.ShapeDtypeStruct((N, DH), jnp.float32),
        jax.ShapeDtypeStruct((N, DH), jnp.float32),
    ),
    mesh=plsc.VectorSubcoreMesh(core_axis_name="c", subcore_axis_name="s"),
    scratch_types=[
        pltpu.VMEM((NCHUNK, CHUNK), jnp.int32),
        pltpu.VMEM((NCHUNK, CHUNK), jnp.int32),
        pltpu.VMEM((NCHUNK, CHUNK), jnp.int32),
        pltpu.VMEM((NBUF, CHUNK, DH), jnp.float32),
        pltpu.VMEM_SHARED((AROWS, DH), jnp.float32),
        pltpu.SemaphoreType.DMA,
        pltpu.SemaphoreType.DMA,
        pltpu.SemaphoreType.DMA,
        pltpu.SemaphoreType.DMA,
    ],
    name="sage_sc_agg",
)


def _sc_deg_body(dst_h, zd, ones_h, deg_out, dst_v, ones_v, deg_s):
  cid = lax.axis_index("c")
  sid = lax.axis_index("s")

  pltpu.sync_copy(dst_h.at[sid], dst_v)
  pltpu.sync_copy(ones_h, ones_v)

  r0 = pl.multiple_of(sid * DROWS_PT, 8)
  t0 = NTILES * DROWS_PT
  last = sid == NTILES - 1

  @pl.when(cid == 0)
  def _():
    pltpu.sync_copy(zd, deg_s.at[pl.ds(r0, DROWS_PT)])

    @pl.when(last)
    def _():
      pltpu.sync_copy(zd.at[pl.ds(0, DROWS_TAIL)],
                      deg_s.at[pl.ds(t0, DROWS_TAIL)])

  plsc.subcore_barrier()

  @pl.when(cid == 0)
  def _():
    def body(j, carry):
      pltpu.sync_copy(ones_v, deg_s.at[dst_v.at[j]], add=True)
      return carry

    lax.fori_loop(0, NCHUNK, body, 0)

  plsc.subcore_barrier()

  @pl.when(cid == 0)
  def _():
    pltpu.sync_copy(deg_s.at[pl.ds(r0, DROWS_PT)],
                    deg_out.at[pl.ds(r0, DROWS_PT)])

    @pl.when(last)
    def _():
      pltpu.sync_copy(deg_s.at[pl.ds(t0, DWB_TAIL)],
                      deg_out.at[pl.ds(t0, DWB_TAIL)])


_sc_deg = pl.kernel(
    _sc_deg_body,
    out_type=jax.ShapeDtypeStruct((N, DEGW), jnp.float32),
    mesh=plsc.VectorSubcoreMesh(core_axis_name="c", subcore_axis_name="s"),
    scratch_types=[
        pltpu.VMEM((NCHUNK, CHUNK), jnp.int32),
        pltpu.VMEM((CHUNK, DEGW), jnp.float32),
        pltpu.VMEM_SHARED((DROWS, DEGW), jnp.float32),
    ],
    name="sage_sc_deg",
)


RB = 1000


def _dense_body(x_ref, alo_ref, ahi_ref, deg_ref, wsT_ref, wnT_ref,
                bias_ref, o_ref):
  r = 1.0 / jnp.maximum(deg_ref[...][:, 0:1], 1.0)
  h = jnp.concatenate([alo_ref[...], ahi_ref[...]], axis=1) * r
  o_ref[...] = (
      jnp.dot(x_ref[...], wsT_ref[...], preferred_element_type=jnp.float32,
              precision=jax.lax.Precision.HIGHEST)
      + jnp.dot(h, wnT_ref[...], preferred_element_type=jnp.float32,
                precision=jax.lax.Precision.HIGHEST)
      + bias_ref[...]
  )


def _dense(feat, alo, ahi, deg, wsT, wnT, bias):
  grid = (N // RB,)
  return pl.pallas_call(
      _dense_body,
      grid=grid,
      in_specs=[
          pl.BlockSpec((RB, D), lambda i: (i, 0)),
          pl.BlockSpec((RB, DH), lambda i: (i, 0)),
          pl.BlockSpec((RB, DH), lambda i: (i, 0)),
          pl.BlockSpec((RB, DEGW), lambda i: (i, 0)),
          pl.BlockSpec((D, D), lambda i: (0, 0)),
          pl.BlockSpec((D, D), lambda i: (0, 0)),
          pl.BlockSpec((1, D), lambda i: (0, 0)),
      ],
      out_specs=pl.BlockSpec((RB, D), lambda i: (i, 0)),
      out_shape=jax.ShapeDtypeStruct((N, D), jnp.float32),
  )(feat, alo, ahi, deg, wsT, wnT, bias)


def kernel(feat, edge_index, W_self, b_self, W_neigh, b_neigh):
  pad = EPT_PAD - EPT
  src = edge_index[0].astype(jnp.int32).reshape(NTILES, EPT)
  dst = edge_index[1].astype(jnp.int32).reshape(NTILES, EPT)
  src = jnp.pad(src, ((0, 0), (0, pad))).reshape(NTILES, NCHUNK, CHUNK)
  dst = jnp.pad(dst, ((0, 0), (0, pad)),
                constant_values=N).reshape(NTILES, NCHUNK, CHUNK)
  feat_lo = feat[:, :DH]
  feat_hi = feat[:, DH:]
  zf = jnp.zeros((ROWS_PT, DH), jnp.float32)
  zd = jnp.zeros((DROWS_PT, DEGW), jnp.float32)

  ones_h = jnp.ones((CHUNK, DEGW), jnp.float32)
  alo, ahi = _sc_agg(feat_lo, feat_hi, src, dst, zf)
  deg = _sc_deg(dst, zd, ones_h)

  wsT = W_self.T
  wnT = W_neigh.T
  bias = (b_self + b_neigh).reshape(1, D)
  return _dense(feat, alo, ahi, deg, wsT, wnT, bias)


# speedup vs baseline: 3.7986x; 2.0469x over previous
import jax
import jax.numpy as jnp
from jax import lax
from jax.experimental import pallas as pl
from jax.experimental.pallas import tpu as pltpu
from jax.experimental.pallas import tpu_sc as plsc

N = 10000
E = 160000
D = 256
DH = D // 2
NTILES = 16
EPT = E // NTILES
CHUNK = 128
NCHUNK = 80
EPT_PAD = NCHUNK * CHUNK
NBUF = 2
TRASH = 112
AROWS = N + TRASH
Z_PT = AROWS // NTILES
ROWS_PT = 624
WB_TAIL = N - NTILES * ROWS_PT
DEGW = 128


def _sc_agg_body(feat_lo, feat_hi, src_h, dst_h, zf,
                 out_lo, out_hi,
                 srcb, dstb, rows_v, acc_s,
                 si0, si1, di0, di1, g0, g1):
  cid = lax.axis_index("c")
  sid = lax.axis_index("s")
  sem_i = [si0, si1]
  sem_d = [di0, di1]
  sem_g = [g0, g1]

  z0 = pl.multiple_of(sid * Z_PT, 8)
  pltpu.sync_copy(zf, acc_s.at[pl.ds(z0, Z_PT)])

  plsc.subcore_barrier()

  def start_idx(j, b):
    pltpu.async_copy(src_h.at[sid, j], srcb.at[b], sem_i[b])
    pltpu.async_copy(dst_h.at[sid, j], dstb.at[b], sem_d[b])

  def wait_src(j, b):
    pltpu.make_async_copy(src_h.at[sid, j], srcb.at[b], sem_i[b]).wait()

  def wait_dst(j, b):
    pltpu.make_async_copy(dst_h.at[sid, j], dstb.at[b], sem_d[b]).wait()

  def start_gather(j, b):
    @pl.when(cid == 0)
    def _():
      pltpu.async_copy(feat_lo.at[srcb.at[b]], rows_v.at[b], sem_g[b])

    @pl.when(cid != 0)
    def _():
      pltpu.async_copy(feat_hi.at[srcb.at[b]], rows_v.at[b], sem_g[b])

  def wait_gather(j, b):
    @pl.when(cid == 0)
    def _():
      pltpu.make_async_copy(feat_lo.at[srcb.at[b]], rows_v.at[b],
                            sem_g[b]).wait()

    @pl.when(cid != 0)
    def _():
      pltpu.make_async_copy(feat_hi.at[srcb.at[b]], rows_v.at[b],
                            sem_g[b]).wait()

  start_idx(0, 0)
  start_idx(1, 1)
  wait_src(0, 0)
  start_gather(0, 0)

  def step(j, b, b1):
    @pl.when(j + 1 < NCHUNK)
    def _():
      wait_src(j + 1, b1)
      start_gather(j + 1, b1)

    wait_gather(j, b)
    wait_dst(j, b)
    pltpu.sync_copy(rows_v.at[b], acc_s.at[dstb.at[b]], add=True)

    @pl.when(j + NBUF < NCHUNK)
    def _():
      start_idx(j + NBUF, b)

  def outer(g, carry):
    for b in range(NBUF):
      step(g * NBUF + b, b, (b + 1) % NBUF)
    return carry

  lax.fori_loop(0, NCHUNK // NBUF, outer, 0)

  plsc.subcore_barrier()

  r0 = pl.multiple_of(sid * ROWS_PT, 8)
  t0 = NTILES * ROWS_PT
  last = sid == NTILES - 1

  @pl.when(cid == 0)
  def _():
    pltpu.sync_copy(acc_s.at[pl.ds(r0, ROWS_PT)],
                    out_lo.at[pl.ds(r0, ROWS_PT)])

    @pl.when(last)
    def _():
      pltpu.sync_copy(acc_s.at[pl.ds(t0, WB_TAIL)],
                      out_lo.at[pl.ds(t0, WB_TAIL)])

  @pl.when(cid != 0)
  def _():
    pltpu.sync_copy(acc_s.at[pl.ds(r0, ROWS_PT)],
                    out_hi.at[pl.ds(r0, ROWS_PT)])

    @pl.when(last)
    def _():
      pltpu.sync_copy(acc_s.at[pl.ds(t0, WB_TAIL)],
                      out_hi.at[pl.ds(t0, WB_TAIL)])


_sc_agg = pl.kernel(
    _sc_agg_body,
    out_type=(
        jax.ShapeDtypeStruct((N, DH), jnp.float32),
        jax.ShapeDtypeStruct((N, DH), jnp.float32),
    ),
    mesh=plsc.VectorSubcoreMesh(core_axis_name="c", subcore_axis_name="s"),
    scratch_types=[
        pltpu.VMEM((NBUF, CHUNK), jnp.int32),
        pltpu.VMEM((NBUF, CHUNK), jnp.int32),
        pltpu.VMEM((NBUF, CHUNK, DH), jnp.float32),
        pltpu.VMEM_SHARED((AROWS, DH), jnp.float32),
        pltpu.SemaphoreType.DMA,
        pltpu.SemaphoreType.DMA,
        pltpu.SemaphoreType.DMA,
        pltpu.SemaphoreType.DMA,
        pltpu.SemaphoreType.DMA,
        pltpu.SemaphoreType.DMA,
    ],
    name="sage_sc_agg",
)


def _sc_deg_body(dst_h, zd, ones_h, deg_out, dst_v, ones_v, deg_s):
  cid = lax.axis_index("c")
  sid = lax.axis_index("s")

  pltpu.sync_copy(dst_h.at[sid], dst_v)
  pltpu.sync_copy(ones_h, ones_v)

  z0 = pl.multiple_of(sid * Z_PT, 8)

  @pl.when(cid == 0)
  def _():
    pltpu.sync_copy(zd, deg_s.at[pl.ds(z0, Z_PT)])

  plsc.subcore_barrier()

  @pl.when(cid == 0)
  def _():
    def body(j, carry):
      pltpu.sync_copy(ones_v, deg_s.at[dst_v.at[j]], add=True)
      return carry

    lax.fori_loop(0, NCHUNK, body, 0)

  plsc.subcore_barrier()

  r0 = pl.multiple_of(sid * ROWS_PT, 8)
  t0 = NTILES * ROWS_PT
  last = sid == NTILES - 1

  @pl.when(cid == 0)
  def _():
    pltpu.sync_copy(deg_s.at[pl.ds(r0, ROWS_PT)],
                    deg_out.at[pl.ds(r0, ROWS_PT)])

    @pl.when(last)
    def _():
      pltpu.sync_copy(deg_s.at[pl.ds(t0, WB_TAIL)],
                      deg_out.at[pl.ds(t0, WB_TAIL)])


_sc_deg = pl.kernel(
    _sc_deg_body,
    out_type=jax.ShapeDtypeStruct((N, DEGW), jnp.float32),
    mesh=plsc.VectorSubcoreMesh(core_axis_name="c", subcore_axis_name="s"),
    scratch_types=[
        pltpu.VMEM((NCHUNK, CHUNK), jnp.int32),
        pltpu.VMEM((CHUNK, DEGW), jnp.float32),
        pltpu.VMEM_SHARED((AROWS, DEGW), jnp.float32),
    ],
    name="sage_sc_deg",
)


RB = 1000


def _dense_body(x_ref, alo_ref, ahi_ref, deg_ref, wsT_ref, wnT_ref,
                bias_ref, o_ref):
  r = 1.0 / jnp.maximum(deg_ref[...][:, 0:1], 1.0)
  h = jnp.concatenate([alo_ref[...], ahi_ref[...]], axis=1) * r
  o_ref[...] = (
      jnp.dot(x_ref[...], wsT_ref[...], preferred_element_type=jnp.float32,
              precision=jax.lax.Precision.HIGHEST)
      + jnp.dot(h, wnT_ref[...], preferred_element_type=jnp.float32,
                precision=jax.lax.Precision.HIGHEST)
      + bias_ref[...]
  )


def _dense(feat, alo, ahi, deg, wsT, wnT, bias):
  grid = (N // RB,)
  return pl.pallas_call(
      _dense_body,
      grid=grid,
      in_specs=[
          pl.BlockSpec((RB, D), lambda i: (i, 0)),
          pl.BlockSpec((RB, DH), lambda i: (i, 0)),
          pl.BlockSpec((RB, DH), lambda i: (i, 0)),
          pl.BlockSpec((RB, DEGW), lambda i: (i, 0)),
          pl.BlockSpec((D, D), lambda i: (0, 0)),
          pl.BlockSpec((D, D), lambda i: (0, 0)),
          pl.BlockSpec((1, D), lambda i: (0, 0)),
      ],
      out_specs=pl.BlockSpec((RB, D), lambda i: (i, 0)),
      out_shape=jax.ShapeDtypeStruct((N, D), jnp.float32),
  )(feat, alo, ahi, deg, wsT, wnT, bias)


def kernel(feat, edge_index, W_self, b_self, W_neigh, b_neigh):
  pad = EPT_PAD - EPT
  src = edge_index[0].astype(jnp.int32).reshape(NTILES, EPT)
  dst = edge_index[1].astype(jnp.int32).reshape(NTILES, EPT)
  src = jnp.pad(src, ((0, 0), (0, pad))).reshape(NTILES, NCHUNK, CHUNK)
  dst = jnp.pad(dst, ((0, 0), (0, pad)),
                constant_values=N).reshape(NTILES, NCHUNK, CHUNK)
  feat_lo = feat[:, :DH]
  feat_hi = feat[:, DH:]
  zf = jnp.zeros((Z_PT, DH), jnp.float32)
  zd = jnp.zeros((Z_PT, DEGW), jnp.float32)
  ones_h = jnp.ones((CHUNK, DEGW), jnp.float32)

  alo, ahi = _sc_agg(feat_lo, feat_hi, src, dst, zf)
  deg = _sc_deg(dst, zd, ones_h)

  wsT = W_self.T
  wnT = W_neigh.T
  bias = (b_self + b_neigh).reshape(1, D)
  return _dense(feat, alo, ahi, deg, wsT, wnT, bias)


# speedup vs baseline: 4.0510x; 1.0665x over previous
import jax
import jax.numpy as jnp
from jax import lax
from jax.experimental import pallas as pl
from jax.experimental.pallas import tpu as pltpu
from jax.experimental.pallas import tpu_sc as plsc

N = 10000
E = 160000
D = 256
DH = D // 2
NTILES = 16
EPT = E // NTILES
CHUNK = 128
NCHUNK = 80
EPT_PAD = NCHUNK * CHUNK
NBUF = 2
TRASH = 112
AROWS = N + TRASH
Z_PT = AROWS // NTILES
ROWS_PT = 624
WB_TAIL = N - NTILES * ROWS_PT
DEGW = 128


def _sc_agg_body(feat_lo, feat_hi, src_h, dst_h, zf,
                 out_lo, out_hi,
                 srcb, dstb, rows_v, acc_s,
                 si0, si1, di0, di1, g0, g1):
  cid = lax.axis_index("c")
  sid = lax.axis_index("s")
  sem_i = [si0, si1]
  sem_d = [di0, di1]
  sem_g = [g0, g1]

  z0 = pl.multiple_of(sid * Z_PT, 8)
  pltpu.sync_copy(zf, acc_s.at[pl.ds(z0, Z_PT)])

  plsc.subcore_barrier()

  def start_idx(j, b):
    pltpu.async_copy(src_h.at[sid, j], srcb.at[b], sem_i[b])
    pltpu.async_copy(dst_h.at[sid, j], dstb.at[b], sem_d[b])

  def wait_src(j, b):
    pltpu.make_async_copy(src_h.at[sid, j], srcb.at[b], sem_i[b]).wait()

  def wait_dst(j, b):
    pltpu.make_async_copy(dst_h.at[sid, j], dstb.at[b], sem_d[b]).wait()

  def start_gather(j, b):
    @pl.when(cid == 0)
    def _():
      pltpu.async_copy(feat_lo.at[srcb.at[b]], rows_v.at[b], sem_g[b])

    @pl.when(cid != 0)
    def _():
      pltpu.async_copy(feat_hi.at[srcb.at[b]], rows_v.at[b], sem_g[b])

  def wait_gather(j, b):
    @pl.when(cid == 0)
    def _():
      pltpu.make_async_copy(feat_lo.at[srcb.at[b]], rows_v.at[b],
                            sem_g[b]).wait()

    @pl.when(cid != 0)
    def _():
      pltpu.make_async_copy(feat_hi.at[srcb.at[b]], rows_v.at[b],
                            sem_g[b]).wait()

  start_idx(0, 0)
  start_idx(1, 1)
  wait_src(0, 0)
  start_gather(0, 0)

  def step(j, b, b1):
    @pl.when(j + 1 < NCHUNK)
    def _():
      wait_src(j + 1, b1)
      start_gather(j + 1, b1)

    wait_gather(j, b)
    wait_dst(j, b)
    pltpu.sync_copy(rows_v.at[b], acc_s.at[dstb.at[b]], add=True)

    @pl.when(j + NBUF < NCHUNK)
    def _():
      start_idx(j + NBUF, b)

  def outer(g, carry):
    for b in range(NBUF):
      step(g * NBUF + b, b, (b + 1) % NBUF)
    return carry

  lax.fori_loop(0, NCHUNK // NBUF, outer, 0)

  plsc.subcore_barrier()

  r0 = pl.multiple_of(sid * ROWS_PT, 8)
  t0 = NTILES * ROWS_PT
  last = sid == NTILES - 1

  @pl.when(cid == 0)
  def _():
    pltpu.sync_copy(acc_s.at[pl.ds(r0, ROWS_PT)],
                    out_lo.at[pl.ds(r0, ROWS_PT)])

    @pl.when(last)
    def _():
      pltpu.sync_copy(acc_s.at[pl.ds(t0, WB_TAIL)],
                      out_lo.at[pl.ds(t0, WB_TAIL)])

  @pl.when(cid != 0)
  def _():
    pltpu.sync_copy(acc_s.at[pl.ds(r0, ROWS_PT)],
                    out_hi.at[pl.ds(r0, ROWS_PT)])

    @pl.when(last)
    def _():
      pltpu.sync_copy(acc_s.at[pl.ds(t0, WB_TAIL)],
                      out_hi.at[pl.ds(t0, WB_TAIL)])


_sc_agg = pl.kernel(
    _sc_agg_body,
    out_type=(
        jax.ShapeDtypeStruct((N, DH), jnp.float32),
        jax.ShapeDtypeStruct((N, DH), jnp.float32),
    ),
    mesh=plsc.VectorSubcoreMesh(core_axis_name="c", subcore_axis_name="s"),
    scratch_types=[
        pltpu.VMEM((NBUF, CHUNK), jnp.int32),
        pltpu.VMEM((NBUF, CHUNK), jnp.int32),
        pltpu.VMEM((NBUF, CHUNK, DH), jnp.float32),
        pltpu.VMEM_SHARED((AROWS, DH), jnp.float32),
        pltpu.SemaphoreType.DMA,
        pltpu.SemaphoreType.DMA,
        pltpu.SemaphoreType.DMA,
        pltpu.SemaphoreType.DMA,
        pltpu.SemaphoreType.DMA,
        pltpu.SemaphoreType.DMA,
    ],
    name="sage_sc_agg",
)


def _sc_deg_body(dst_h, zd, ones_h, deg0_out, deg1_out, dst_v, ones_v, deg_s):
  cid = lax.axis_index("c")
  sid = lax.axis_index("s")

  pltpu.sync_copy(dst_h.at[sid], dst_v)
  pltpu.sync_copy(ones_h, ones_v)

  z0 = pl.multiple_of(sid * Z_PT, 8)
  pltpu.sync_copy(zd, deg_s.at[pl.ds(z0, Z_PT)])

  plsc.subcore_barrier()

  half = NCHUNK // 2
  base = cid * half

  def body(j, carry):
    pltpu.sync_copy(ones_v, deg_s.at[dst_v.at[base + j]], add=True)
    return carry

  lax.fori_loop(0, half, body, 0)

  plsc.subcore_barrier()

  r0 = pl.multiple_of(sid * ROWS_PT, 8)
  t0 = NTILES * ROWS_PT
  last = sid == NTILES - 1

  @pl.when(cid == 0)
  def _():
    pltpu.sync_copy(deg_s.at[pl.ds(r0, ROWS_PT)],
                    deg0_out.at[pl.ds(r0, ROWS_PT)])

    @pl.when(last)
    def _():
      pltpu.sync_copy(deg_s.at[pl.ds(t0, WB_TAIL)],
                      deg0_out.at[pl.ds(t0, WB_TAIL)])

  @pl.when(cid != 0)
  def _():
    pltpu.sync_copy(deg_s.at[pl.ds(r0, ROWS_PT)],
                    deg1_out.at[pl.ds(r0, ROWS_PT)])

    @pl.when(last)
    def _():
      pltpu.sync_copy(deg_s.at[pl.ds(t0, WB_TAIL)],
                      deg1_out.at[pl.ds(t0, WB_TAIL)])


_sc_deg = pl.kernel(
    _sc_deg_body,
    out_type=(jax.ShapeDtypeStruct((N, DEGW), jnp.float32),
              jax.ShapeDtypeStruct((N, DEGW), jnp.float32)),
    mesh=plsc.VectorSubcoreMesh(core_axis_name="c", subcore_axis_name="s"),
    scratch_types=[
        pltpu.VMEM((NCHUNK, CHUNK), jnp.int32),
        pltpu.VMEM((CHUNK, DEGW), jnp.float32),
        pltpu.VMEM_SHARED((AROWS, DEGW), jnp.float32),
    ],
    name="sage_sc_deg",
)


RB = 1000


def _dense_body(x_ref, alo_ref, ahi_ref, deg0_ref, deg1_ref, wsT_ref,
                wnT_ref, bias_ref, o_ref):
  d = deg0_ref[...][:, 0:1] + deg1_ref[...][:, 0:1]
  r = 1.0 / jnp.maximum(d, 1.0)
  h = jnp.concatenate([alo_ref[...], ahi_ref[...]], axis=1) * r
  o_ref[...] = (
      jnp.dot(x_ref[...], wsT_ref[...], preferred_element_type=jnp.float32,
              precision=jax.lax.Precision.HIGHEST)
      + jnp.dot(h, wnT_ref[...], preferred_element_type=jnp.float32,
                precision=jax.lax.Precision.HIGHEST)
      + bias_ref[...]
  )


def _dense(feat, alo, ahi, deg0, deg1, wsT, wnT, bias):
  grid = (N // RB,)
  return pl.pallas_call(
      _dense_body,
      grid=grid,
      in_specs=[
          pl.BlockSpec((RB, D), lambda i: (i, 0)),
          pl.BlockSpec((RB, DH), lambda i: (i, 0)),
          pl.BlockSpec((RB, DH), lambda i: (i, 0)),
          pl.BlockSpec((RB, DEGW), lambda i: (i, 0)),
          pl.BlockSpec((RB, DEGW), lambda i: (i, 0)),
          pl.BlockSpec((D, D), lambda i: (0, 0)),
          pl.BlockSpec((D, D), lambda i: (0, 0)),
          pl.BlockSpec((1, D), lambda i: (0, 0)),
      ],
      out_specs=pl.BlockSpec((RB, D), lambda i: (i, 0)),
      out_shape=jax.ShapeDtypeStruct((N, D), jnp.float32),
  )(feat, alo, ahi, deg0, deg1, wsT, wnT, bias)


def kernel(feat, edge_index, W_self, b_self, W_neigh, b_neigh):
  pad = EPT_PAD - EPT
  src = edge_index[0].astype(jnp.int32).reshape(NTILES, EPT)
  dst = edge_index[1].astype(jnp.int32).reshape(NTILES, EPT)
  src = jnp.pad(src, ((0, 0), (0, pad))).reshape(NTILES, NCHUNK, CHUNK)
  dst = jnp.pad(dst, ((0, 0), (0, pad)),
                constant_values=N).reshape(NTILES, NCHUNK, CHUNK)
  feat_lo = feat[:, :DH]
  feat_hi = feat[:, DH:]
  zf = jnp.zeros((Z_PT, DH), jnp.float32)
  zd = jnp.zeros((Z_PT, DEGW), jnp.float32)
  ones_h = jnp.ones((CHUNK, DEGW), jnp.float32)

  alo, ahi = _sc_agg(feat_lo, feat_hi, src, dst, zf)
  deg0, deg1 = _sc_deg(dst, zd, ones_h)

  wsT = W_self.T
  wnT = W_neigh.T
  bias = (b_self + b_neigh).reshape(1, D)
  return _dense(feat, alo, ahi, deg0, deg1, wsT, wnT, bias)


# speedup vs baseline: 4.2740x; 1.0550x over previous
import jax
import jax.numpy as jnp
from jax import lax
from jax.experimental import pallas as pl
from jax.experimental.pallas import tpu as pltpu
from jax.experimental.pallas import tpu_sc as plsc

N = 10000
E = 160000
D = 256
DH = D // 2
NTILES = 16
EPT = E // NTILES
CHUNK = 80
NCHUNK = 128
EPT_PAD = NCHUNK * CHUNK
NBUF = 4
TRASH = 112
AROWS = N + TRASH
Z_PT = AROWS // NTILES
ROWS_PT = 624
WB_TAIL = N - NTILES * ROWS_PT
DEGW = 128


def _sc_agg_body(feat_lo, feat_hi, src_h, dst_h, zf,
                 out_lo, out_hi,
                 srcb, dstb, rows_v, acc_s,
                 si0, si1, si2, si3, di0, di1, di2, di3,
                 g0, g1, g2, g3, s0, s1, s2, s3):
  cid = lax.axis_index("c")
  sid = lax.axis_index("s")
  sem_i = [si0, si1, si2, si3]
  sem_d = [di0, di1, di2, di3]
  sem_g = [g0, g1, g2, g3]
  sem_s = [s0, s1, s2, s3]

  z0 = pl.multiple_of(sid * Z_PT, 8)
  pltpu.sync_copy(zf, acc_s.at[pl.ds(z0, Z_PT)])

  plsc.subcore_barrier()

  def start_idx(j, b):
    pltpu.async_copy(src_h.at[sid, j], srcb.at[b], sem_i[b])
    pltpu.async_copy(dst_h.at[sid, j], dstb.at[b], sem_d[b])

  def wait_src(j, b):
    pltpu.make_async_copy(src_h.at[sid, j], srcb.at[b], sem_i[b]).wait()

  def wait_dst(j, b):
    pltpu.make_async_copy(dst_h.at[sid, j], dstb.at[b], sem_d[b]).wait()

  def start_gather(j, b):
    @pl.when(cid == 0)
    def _():
      pltpu.async_copy(feat_lo.at[srcb.at[b]], rows_v.at[b], sem_g[b])

    @pl.when(cid != 0)
    def _():
      pltpu.async_copy(feat_hi.at[srcb.at[b]], rows_v.at[b], sem_g[b])

  def wait_gather(j, b):
    @pl.when(cid == 0)
    def _():
      pltpu.make_async_copy(feat_lo.at[srcb.at[b]], rows_v.at[b],
                            sem_g[b]).wait()

    @pl.when(cid != 0)
    def _():
      pltpu.make_async_copy(feat_hi.at[srcb.at[b]], rows_v.at[b],
                            sem_g[b]).wait()

  def start_scatter(j, b):
    pltpu.async_copy(rows_v.at[b], acc_s.at[dstb.at[b]], sem_s[b], add=True)

  def wait_scatter(j, b):
    pltpu.make_async_copy(rows_v.at[b], acc_s.at[dstb.at[b]],
                          sem_s[b]).wait()

  start_idx(0, 0)
  start_idx(1, 1)
  wait_src(0, 0)
  start_gather(0, 0)

  def step(j, b):
    b1 = (b + 1) % NBUF
    b2 = (b + 2) % NBUF

    @pl.when(j >= 2)
    def _():
      wait_scatter(j - 2, b2)

    @pl.when(j + 2 < NCHUNK)
    def _():
      start_idx(j + 2, b2)

    @pl.when(j + 1 < NCHUNK)
    def _():
      wait_src(j + 1, b1)
      start_gather(j + 1, b1)

    wait_gather(j, b)
    wait_dst(j, b)
    start_scatter(j, b)

  def outer(g, carry):
    for b in range(NBUF):
      step(g * NBUF + b, b)
    return carry

  lax.fori_loop(0, NCHUNK // NBUF, outer, 0)
  wait_scatter(NCHUNK - 2, (NCHUNK - 2) % NBUF)
  wait_scatter(NCHUNK - 1, (NCHUNK - 1) % NBUF)

  plsc.subcore_barrier()

  r0 = pl.multiple_of(sid * ROWS_PT, 8)
  t0 = NTILES * ROWS_PT
  last = sid == NTILES - 1

  @pl.when(cid == 0)
  def _():
    pltpu.sync_copy(acc_s.at[pl.ds(r0, ROWS_PT)],
                    out_lo.at[pl.ds(r0, ROWS_PT)])

    @pl.when(last)
    def _():
      pltpu.sync_copy(acc_s.at[pl.ds(t0, WB_TAIL)],
                      out_lo.at[pl.ds(t0, WB_TAIL)])

  @pl.when(cid != 0)
  def _():
    pltpu.sync_copy(acc_s.at[pl.ds(r0, ROWS_PT)],
                    out_hi.at[pl.ds(r0, ROWS_PT)])

    @pl.when(last)
    def _():
      pltpu.sync_copy(acc_s.at[pl.ds(t0, WB_TAIL)],
                      out_hi.at[pl.ds(t0, WB_TAIL)])


_sc_agg = pl.kernel(
    _sc_agg_body,
    out_type=(
        jax.ShapeDtypeStruct((N, DH), jnp.float32),
        jax.ShapeDtypeStruct((N, DH), jnp.float32),
    ),
    mesh=plsc.VectorSubcoreMesh(core_axis_name="c", subcore_axis_name="s"),
    scratch_types=[
        pltpu.VMEM((NBUF, CHUNK), jnp.int32),
        pltpu.VMEM((NBUF, CHUNK), jnp.int32),
        pltpu.VMEM((NBUF, CHUNK, DH), jnp.float32),
        pltpu.VMEM_SHARED((AROWS, DH), jnp.float32),
    ] + [pltpu.SemaphoreType.DMA] * 16,
    name="sage_sc_agg",
)


def _sc_deg_body(dst_h, zd, ones_h, deg0_out, deg1_out, dst_v, ones_v, deg_s):
  cid = lax.axis_index("c")
  sid = lax.axis_index("s")

  pltpu.sync_copy(dst_h.at[sid], dst_v)
  pltpu.sync_copy(ones_h, ones_v)

  z0 = pl.multiple_of(sid * Z_PT, 8)
  pltpu.sync_copy(zd, deg_s.at[pl.ds(z0, Z_PT)])

  plsc.subcore_barrier()

  half = NCHUNK // 2
  base = cid * half

  def body(j, carry):
    pltpu.sync_copy(ones_v, deg_s.at[dst_v.at[base + j]], add=True)
    return carry

  lax.fori_loop(0, half, body, 0)

  plsc.subcore_barrier()

  r0 = pl.multiple_of(sid * ROWS_PT, 8)
  t0 = NTILES * ROWS_PT
  last = sid == NTILES - 1

  @pl.when(cid == 0)
  def _():
    pltpu.sync_copy(deg_s.at[pl.ds(r0, ROWS_PT)],
                    deg0_out.at[pl.ds(r0, ROWS_PT)])

    @pl.when(last)
    def _():
      pltpu.sync_copy(deg_s.at[pl.ds(t0, WB_TAIL)],
                      deg0_out.at[pl.ds(t0, WB_TAIL)])

  @pl.when(cid != 0)
  def _():
    pltpu.sync_copy(deg_s.at[pl.ds(r0, ROWS_PT)],
                    deg1_out.at[pl.ds(r0, ROWS_PT)])

    @pl.when(last)
    def _():
      pltpu.sync_copy(deg_s.at[pl.ds(t0, WB_TAIL)],
                      deg1_out.at[pl.ds(t0, WB_TAIL)])


_sc_deg = pl.kernel(
    _sc_deg_body,
    out_type=(jax.ShapeDtypeStruct((N, DEGW), jnp.float32),
              jax.ShapeDtypeStruct((N, DEGW), jnp.float32)),
    mesh=plsc.VectorSubcoreMesh(core_axis_name="c", subcore_axis_name="s"),
    scratch_types=[
        pltpu.VMEM((NCHUNK, CHUNK), jnp.int32),
        pltpu.VMEM((CHUNK, DEGW), jnp.float32),
        pltpu.VMEM_SHARED((AROWS, DEGW), jnp.float32),
    ],
    name="sage_sc_deg",
)


RB = 1000


def _dense_body(x_ref, alo_ref, ahi_ref, deg0_ref, deg1_ref, wsT_ref,
                wnT_ref, bias_ref, o_ref):
  d = deg0_ref[...][:, 0:1] + deg1_ref[...][:, 0:1]
  r = 1.0 / jnp.maximum(d, 1.0)
  h = jnp.concatenate([alo_ref[...], ahi_ref[...]], axis=1) * r
  o_ref[...] = (
      jnp.dot(x_ref[...], wsT_ref[...], preferred_element_type=jnp.float32,
              precision=jax.lax.Precision.HIGHEST)
      + jnp.dot(h, wnT_ref[...], preferred_element_type=jnp.float32,
                precision=jax.lax.Precision.HIGHEST)
      + bias_ref[...]
  )


def _dense(feat, alo, ahi, deg0, deg1, wsT, wnT, bias):
  grid = (N // RB,)
  return pl.pallas_call(
      _dense_body,
      grid=grid,
      in_specs=[
          pl.BlockSpec((RB, D), lambda i: (i, 0)),
          pl.BlockSpec((RB, DH), lambda i: (i, 0)),
          pl.BlockSpec((RB, DH), lambda i: (i, 0)),
          pl.BlockSpec((RB, DEGW), lambda i: (i, 0)),
          pl.BlockSpec((RB, DEGW), lambda i: (i, 0)),
          pl.BlockSpec((D, D), lambda i: (0, 0)),
          pl.BlockSpec((D, D), lambda i: (0, 0)),
          pl.BlockSpec((1, D), lambda i: (0, 0)),
      ],
      out_specs=pl.BlockSpec((RB, D), lambda i: (i, 0)),
      out_shape=jax.ShapeDtypeStruct((N, D), jnp.float32),
  )(feat, alo, ahi, deg0, deg1, wsT, wnT, bias)


def kernel(feat, edge_index, W_self, b_self, W_neigh, b_neigh):
  pad = EPT_PAD - EPT
  src = edge_index[0].astype(jnp.int32).reshape(NTILES, EPT)
  dst = edge_index[1].astype(jnp.int32).reshape(NTILES, EPT)
  src = jnp.pad(src, ((0, 0), (0, pad))).reshape(NTILES, NCHUNK, CHUNK)
  dst = jnp.pad(dst, ((0, 0), (0, pad)),
                constant_values=N).reshape(NTILES, NCHUNK, CHUNK)
  feat_lo = feat[:, :DH]
  feat_hi = feat[:, DH:]
  zf = jnp.zeros((Z_PT, DH), jnp.float32)
  zd = jnp.zeros((Z_PT, DEGW), jnp.float32)
  ones_h = jnp.ones((CHUNK, DEGW), jnp.float32)

  alo, ahi = _sc_agg(feat_lo, feat_hi, src, dst, zf)
  deg0, deg1 = _sc_deg(dst, zd, ones_h)

  wsT = W_self.T
  wnT = W_neigh.T
  bias = (b_self + b_neigh).reshape(1, D)
  return _dense(feat, alo, ahi, deg0, deg1, wsT, wnT, bias)


# speedup vs baseline: 4.2968x; 1.0054x over previous
import jax
import jax.numpy as jnp
from jax import lax
from jax.experimental import pallas as pl
from jax.experimental.pallas import tpu as pltpu
from jax.experimental.pallas import tpu_sc as plsc

N = 10000
E = 160000
D = 256
DH = D // 2
NTILES = 16
EPT = E // NTILES
CHUNK = 80
NCHUNK = 128
EPT_PAD = NCHUNK * CHUNK
NBUF = 4
TRASH = 112
AROWS = N + TRASH
Z_PT = AROWS // NTILES
ROWS_PT = 624
WB_TAIL = N - NTILES * ROWS_PT
DEGW = 128


def _sc_agg_body(feat_lo, feat_hi, src_h, dst_h, zf,
                 out_lo, out_hi,
                 srcb, dstb, rows_v, acc_s,
                 si0, si1, si2, si3, di0, di1, di2, di3,
                 g0, g1, g2, g3, s0, s1, s2, s3):
  cid = lax.axis_index("c")
  sid = lax.axis_index("s")
  sem_i = [si0, si1, si2, si3]
  sem_d = [di0, di1, di2, di3]
  sem_g = [g0, g1, g2, g3]
  sem_s = [s0, s1, s2, s3]

  z0 = pl.multiple_of(sid * Z_PT, 8)
  pltpu.sync_copy(zf, acc_s.at[pl.ds(z0, Z_PT)])

  plsc.subcore_barrier()

  def start_idx(j, b):
    pltpu.async_copy(src_h.at[sid, j], srcb.at[b], sem_i[b])
    pltpu.async_copy(dst_h.at[sid, j], dstb.at[b], sem_d[b])

  def wait_src(j, b):
    pltpu.make_async_copy(src_h.at[sid, j], srcb.at[b], sem_i[b]).wait()

  def wait_dst(j, b):
    pltpu.make_async_copy(dst_h.at[sid, j], dstb.at[b], sem_d[b]).wait()

  def start_gather(j, b):
    @pl.when(cid == 0)
    def _():
      pltpu.async_copy(feat_lo.at[srcb.at[b]], rows_v.at[b], sem_g[b])

    @pl.when(cid != 0)
    def _():
      pltpu.async_copy(feat_hi.at[srcb.at[b]], rows_v.at[b], sem_g[b])

  def wait_gather(j, b):
    @pl.when(cid == 0)
    def _():
      pltpu.make_async_copy(feat_lo.at[srcb.at[b]], rows_v.at[b],
                            sem_g[b]).wait()

    @pl.when(cid != 0)
    def _():
      pltpu.make_async_copy(feat_hi.at[srcb.at[b]], rows_v.at[b],
                            sem_g[b]).wait()

  def start_scatter(j, b):
    pltpu.async_copy(rows_v.at[b], acc_s.at[dstb.at[b]], sem_s[b], add=True)

  def wait_scatter(j, b):
    pltpu.make_async_copy(rows_v.at[b], acc_s.at[dstb.at[b]],
                          sem_s[b]).wait()

  start_idx(0, 0)
  start_idx(1, 1)
  wait_src(0, 0)
  start_gather(0, 0)

  def step(j, b):
    b1 = (b + 1) % NBUF
    b2 = (b + 2) % NBUF

    @pl.when(j >= 2)
    def _():
      wait_scatter(j - 2, b2)

    @pl.when(j + 2 < NCHUNK)
    def _():
      start_idx(j + 2, b2)

    @pl.when(j + 1 < NCHUNK)
    def _():
      wait_src(j + 1, b1)
      start_gather(j + 1, b1)

    wait_gather(j, b)
    wait_dst(j, b)
    start_scatter(j, b)

  def outer(g, carry):
    for b in range(NBUF):
      step(g * NBUF + b, b)
    return carry

  lax.fori_loop(0, NCHUNK // NBUF, outer, 0)
  wait_scatter(NCHUNK - 2, (NCHUNK - 2) % NBUF)
  wait_scatter(NCHUNK - 1, (NCHUNK - 1) % NBUF)

  plsc.subcore_barrier()

  r0 = pl.multiple_of(sid * ROWS_PT, 8)
  t0 = NTILES * ROWS_PT
  last = sid == NTILES - 1

  @pl.when(cid == 0)
  def _():
    pltpu.sync_copy(acc_s.at[pl.ds(r0, ROWS_PT)],
                    out_lo.at[pl.ds(r0, ROWS_PT)])

    @pl.when(last)
    def _():
      pltpu.sync_copy(acc_s.at[pl.ds(t0, WB_TAIL)],
                      out_lo.at[pl.ds(t0, WB_TAIL)])

  @pl.when(cid != 0)
  def _():
    pltpu.sync_copy(acc_s.at[pl.ds(r0, ROWS_PT)],
                    out_hi.at[pl.ds(r0, ROWS_PT)])

    @pl.when(last)
    def _():
      pltpu.sync_copy(acc_s.at[pl.ds(t0, WB_TAIL)],
                      out_hi.at[pl.ds(t0, WB_TAIL)])


_sc_agg = pl.kernel(
    _sc_agg_body,
    out_type=(
        jax.ShapeDtypeStruct((N, DH), jnp.float32),
        jax.ShapeDtypeStruct((N, DH), jnp.float32),
    ),
    mesh=plsc.VectorSubcoreMesh(core_axis_name="c", subcore_axis_name="s"),
    scratch_types=[
        pltpu.VMEM((NBUF, CHUNK), jnp.int32),
        pltpu.VMEM((NBUF, CHUNK), jnp.int32),
        pltpu.VMEM((NBUF, CHUNK, DH), jnp.float32),
        pltpu.VMEM_SHARED((AROWS, DH), jnp.float32),
    ] + [pltpu.SemaphoreType.DMA] * 16,
    name="sage_sc_agg",
)


def _sc_deg_body(dst_h, zd, ones_h, deg0_out, deg1_out, dst_v, ones_v, deg_s):
  cid = lax.axis_index("c")
  sid = lax.axis_index("s")

  pltpu.sync_copy(dst_h.at[sid], dst_v)
  pltpu.sync_copy(ones_h, ones_v)

  z0 = pl.multiple_of(sid * Z_PT, 8)
  pltpu.sync_copy(zd, deg_s.at[pl.ds(z0, Z_PT)])

  plsc.subcore_barrier()

  half = NCHUNK // 2
  base = cid * half

  def body(j, carry):
    pltpu.sync_copy(ones_v, deg_s.at[dst_v.at[base + j]], add=True)
    return carry

  lax.fori_loop(0, half, body, 0)

  plsc.subcore_barrier()

  r0 = pl.multiple_of(sid * ROWS_PT, 8)
  t0 = NTILES * ROWS_PT
  last = sid == NTILES - 1

  @pl.when(cid == 0)
  def _():
    pltpu.sync_copy(deg_s.at[pl.ds(r0, ROWS_PT)],
                    deg0_out.at[pl.ds(r0, ROWS_PT)])

    @pl.when(last)
    def _():
      pltpu.sync_copy(deg_s.at[pl.ds(t0, WB_TAIL)],
                      deg0_out.at[pl.ds(t0, WB_TAIL)])

  @pl.when(cid != 0)
  def _():
    pltpu.sync_copy(deg_s.at[pl.ds(r0, ROWS_PT)],
                    deg1_out.at[pl.ds(r0, ROWS_PT)])

    @pl.when(last)
    def _():
      pltpu.sync_copy(deg_s.at[pl.ds(t0, WB_TAIL)],
                      deg1_out.at[pl.ds(t0, WB_TAIL)])


_sc_deg = pl.kernel(
    _sc_deg_body,
    out_type=(jax.ShapeDtypeStruct((N, DEGW), jnp.float32),
              jax.ShapeDtypeStruct((N, DEGW), jnp.float32)),
    mesh=plsc.VectorSubcoreMesh(core_axis_name="c", subcore_axis_name="s"),
    scratch_types=[
        pltpu.VMEM((NCHUNK, CHUNK), jnp.int32),
        pltpu.VMEM((CHUNK, DEGW), jnp.float32),
        pltpu.VMEM_SHARED((AROWS, DEGW), jnp.float32),
    ],
    name="sage_sc_deg",
)


RB = 1000


def _self_body(x_ref, wsT_ref, bias_ref, o_ref):
  o_ref[...] = jnp.dot(
      x_ref[...], wsT_ref[...], preferred_element_type=jnp.float32,
      precision=jax.lax.Precision.HIGHEST) + bias_ref[...]


def _self_mm(feat, wsT, bias):
  return pl.pallas_call(
      _self_body,
      grid=(N // RB,),
      in_specs=[
          pl.BlockSpec((RB, D), lambda i: (i, 0)),
          pl.BlockSpec((D, D), lambda i: (0, 0)),
          pl.BlockSpec((1, D), lambda i: (0, 0)),
      ],
      out_specs=pl.BlockSpec((RB, D), lambda i: (i, 0)),
      out_shape=jax.ShapeDtypeStruct((N, D), jnp.float32),
  )(feat, wsT, bias)


def _dense_body(s_ref, alo_ref, ahi_ref, deg0_ref, deg1_ref,
                wnT_ref, o_ref):
  d = deg0_ref[...][:, 0:1] + deg1_ref[...][:, 0:1]
  r = 1.0 / jnp.maximum(d, 1.0)
  h = jnp.concatenate([alo_ref[...], ahi_ref[...]], axis=1) * r
  o_ref[...] = s_ref[...] + jnp.dot(
      h, wnT_ref[...], preferred_element_type=jnp.float32,
      precision=jax.lax.Precision.HIGHEST)


def _dense(selfmm, alo, ahi, deg0, deg1, wnT):
  grid = (N // RB,)
  return pl.pallas_call(
      _dense_body,
      grid=grid,
      in_specs=[
          pl.BlockSpec((RB, D), lambda i: (i, 0)),
          pl.BlockSpec((RB, DH), lambda i: (i, 0)),
          pl.BlockSpec((RB, DH), lambda i: (i, 0)),
          pl.BlockSpec((RB, DEGW), lambda i: (i, 0)),
          pl.BlockSpec((RB, DEGW), lambda i: (i, 0)),
          pl.BlockSpec((D, D), lambda i: (0, 0)),
      ],
      out_specs=pl.BlockSpec((RB, D), lambda i: (i, 0)),
      out_shape=jax.ShapeDtypeStruct((N, D), jnp.float32),
  )(selfmm, alo, ahi, deg0, deg1, wnT)


def kernel(feat, edge_index, W_self, b_self, W_neigh, b_neigh):
  pad = EPT_PAD - EPT
  src = edge_index[0].astype(jnp.int32).reshape(NTILES, EPT)
  dst = edge_index[1].astype(jnp.int32).reshape(NTILES, EPT)
  src = jnp.pad(src, ((0, 0), (0, pad))).reshape(NTILES, NCHUNK, CHUNK)
  dst = jnp.pad(dst, ((0, 0), (0, pad)),
                constant_values=N).reshape(NTILES, NCHUNK, CHUNK)
  feat_lo = feat[:, :DH]
  feat_hi = feat[:, DH:]
  zf = jnp.zeros((Z_PT, DH), jnp.float32)
  zd = jnp.zeros((Z_PT, DEGW), jnp.float32)
  ones_h = jnp.ones((CHUNK, DEGW), jnp.float32)

  wsT = W_self.T
  wnT = W_neigh.T
  bias = (b_self + b_neigh).reshape(1, D)

  selfmm = _self_mm(feat, wsT, bias)
  alo, ahi = _sc_agg(feat_lo, feat_hi, src, dst, zf)
  deg0, deg1 = _sc_deg(dst, zd, ones_h)
  return _dense(selfmm, alo, ahi, deg0, deg1, wnT)
